# baseline (speedup 1.0000x reference)
import functools

import jax
import jax.numpy as jnp
from jax import lax
from jax.experimental import pallas as pl
from jax.experimental.pallas import tpu as pltpu
from jax.experimental.pallas import tpu_sc as plsc

B = 16384
L = 20
D = 32
NC = 2
NS = 16
NW = NC * NS
BW = B // NW
CS = 32
NCHUNK = BW // CS
ROWS = CS * L
SUB = 128
NSUB = ROWS // SUB
ACC = 520


def _sc_text_body(gt_idx, cat_idx, mech_idx,
                  gt_table, cat_table, mech_table,
                  out,
                  tidx_v, rows0_v, rows1_v, pool_v,
                  gsrc0_v, gdst0_v, gsrc1_v, gdst1_v, recip_v,
                  acc_v, gsem0, gsem1, ssem0, ssem1):
    c = lax.axis_index("c")
    s = lax.axis_index("s")
    wid = c * NS + s
    base = wid * BW
    iota = lax.iota(jnp.int32, 16)

    def zero_pool(_unused):
        def zero_body(i, carry):
            zeros = jnp.zeros((16,), jnp.float32)
            pool_v[i, pl.ds(0, 16)] = zeros
            pool_v[i, pl.ds(16, 16)] = zeros
            return carry
        lax.fori_loop(0, BW, zero_body, 0)

    acc_base = s * ACC
    trash = acc_base + BW

    acc = acc_v
    for (idx_hbm, table, out_col) in (
            (gt_idx, gt_table, 0),
            (cat_idx, cat_table, D),
            (mech_idx, mech_table, 2 * D)):
        zero_pool(None)
        pltpu.sync_copy(pool_v, acc.at[pl.ds(s * ACC, BW)])
        pltpu.sync_copy(idx_hbm.at[pl.ds(base * L, BW * L)], tidx_v)

        def build(chunk, gsrc, gdst):
            for half in range(CS // 16):
                rows16 = chunk * CS + half * 16 + iota
                cnt = jnp.zeros((16,), jnp.int32)
                dst_ok = acc_base + rows16
                pos0 = rows16 * L
                for l in range(L):
                    v = plsc.load_gather(tidx_v, [pos0 + l])
                    m = v != 0
                    cnt = cnt + jnp.where(m, 1, 0).astype(jnp.int32)
                    dst = jnp.where(m, dst_ok, trash)
                    q = l * CS + half * 16
                    gsrc[q // SUB, pl.ds(q % SUB, 16)] = v
                    gdst[q // SUB, pl.ds(q % SUB, 16)] = dst
                r = 1.0 / jnp.maximum(cnt.astype(jnp.float32), 1.0)
                recip_v[pl.ds(chunk * CS + half * 16, 16)] = r

        def fire_gather(gsrc, rows, sem):
            for j in range(NSUB):
                pltpu.async_copy(table.at[gsrc.at[j]],
                                 rows.at[pl.ds(j * SUB, SUB)], sem)

        def drain(rows, sem):
            for j in range(NSUB):
                pltpu.make_async_copy(table.at[pl.ds(0, SUB)],
                                      rows.at[pl.ds(j * SUB, SUB)],
                                      sem).wait()

        def fire_scatter(rows, gdst, sem):
            for j in range(NSUB):
                pltpu.async_copy(rows.at[pl.ds(j * SUB, SUB)],
                                 acc.at[gdst.at[j]], sem, add=True)

        build(0, gsrc0_v, gdst0_v)
        fire_gather(gsrc0_v, rows0_v, gsem0)

        def chunk_body(k, carry):
            a = 2 * k

            @pl.when(k > 0)
            def _():
                drain(rows1_v, ssem1)
            build(a + 1, gsrc1_v, gdst1_v)
            fire_gather(gsrc1_v, rows1_v, gsem1)

            drain(rows0_v, gsem0)
            fire_scatter(rows0_v, gdst0_v, ssem0)
            drain(rows0_v, ssem0)

            @pl.when(k < NCHUNK // 2 - 1)
            def _():
                build(a + 2, gsrc0_v, gdst0_v)
                fire_gather(gsrc0_v, rows0_v, gsem0)

            drain(rows1_v, gsem1)
            fire_scatter(rows1_v, gdst1_v, ssem1)
            return carry
        lax.fori_loop(0, NCHUNK // 2, chunk_body, 0)
        drain(rows1_v, ssem1)

        pltpu.sync_copy(acc.at[pl.ds(acc_base, BW)], pool_v)

        def fix_body(g, carry):
            rv = recip_v[pl.ds(g * 16, 16)]
            for j in range(16):
                i = g * 16 + j
                r = rv[j]
                pool_v[i, pl.ds(0, 16)] = pool_v[i, pl.ds(0, 16)] * r
                pool_v[i, pl.ds(16, 16)] = pool_v[i, pl.ds(16, 16)] * r
            return carry
        lax.fori_loop(0, BW // 16, fix_body, 0)
        pltpu.sync_copy(pool_v, out.at[pl.ds(base, BW), pl.ds(out_col, D)])


def _sc_small_body(bgg_id, complexity, bgg_table, cplx_table, text, out,
                   pool_v, idxa_v, gsem):
    c = lax.axis_index("c")
    s = lax.axis_index("s")
    wid = c * NS + s
    base = wid * BW

    pltpu.sync_copy(text.at[pl.ds(base, BW), pl.ds(0, 3 * D)],
                    out.at[pl.ds(base, BW), pl.ds(2 * D, 3 * D)])

    for (ids, table, col) in ((bgg_id, bgg_table, 0),
                              (complexity, cplx_table, D)):
        pltpu.sync_copy(ids.at[pl.ds(base, BW)], idxa_v)
        descs = []
        for j in range(BW // SUB):
            descs.append(pltpu.async_copy(
                table.at[idxa_v.at[pl.ds(j * SUB, SUB)]],
                pool_v.at[pl.ds(j * SUB, SUB)], gsem))
        for dsc in descs:
            dsc.wait()
        pltpu.sync_copy(pool_v, out.at[pl.ds(base, BW), pl.ds(col, D)])


@functools.partial(jax.jit, static_argnums=())
def kernel(bgg_id, complexity, game_type, category, mechanic,
           bgg_table, complexity_table, gt_table, cat_table, mech_table,
           norm_mean, norm_var):
    cplx_norm = ((complexity_table - norm_mean[None, :])
                 / jnp.sqrt(norm_var[None, :] + 1e-6))

    mesh = plsc.VectorSubcoreMesh(core_axis_name="c", subcore_axis_name="s")
    cp = pltpu.CompilerParams(
        use_tc_tiling_on_sc=False, needs_layout_passes=False)
    run_text = pl.kernel(
        _sc_text_body,
        out_type=jax.ShapeDtypeStruct((B, 3 * D), jnp.float32),
        mesh=mesh,
        compiler_params=cp,
        scratch_types=[
            pltpu.VMEM((BW * L,), jnp.int32),
            pltpu.VMEM((ROWS, D), jnp.float32),
            pltpu.VMEM((ROWS, D), jnp.float32),
            pltpu.VMEM((BW, D), jnp.float32),
            pltpu.VMEM((NSUB, SUB), jnp.int32),
            pltpu.VMEM((NSUB, SUB), jnp.int32),
            pltpu.VMEM((NSUB, SUB), jnp.int32),
            pltpu.VMEM((NSUB, SUB), jnp.int32),
            pltpu.VMEM((BW,), jnp.float32),
            pltpu.VMEM_SHARED((NS * ACC, D), jnp.float32),
            pltpu.SemaphoreType.DMA,
            pltpu.SemaphoreType.DMA,
            pltpu.SemaphoreType.DMA,
            pltpu.SemaphoreType.DMA,
        ],
    )
    run_small = pl.kernel(
        _sc_small_body,
        out_type=jax.ShapeDtypeStruct((B, 5 * D), jnp.float32),
        mesh=mesh,
        compiler_params=cp,
        scratch_types=[
            pltpu.VMEM((BW, D), jnp.float32),
            pltpu.VMEM((BW,), jnp.int32),
            pltpu.SemaphoreType.DMA,
        ],
    )
    text = run_text(game_type.astype(jnp.int32).reshape(-1),
                    category.astype(jnp.int32).reshape(-1),
                    mechanic.astype(jnp.int32).reshape(-1),
                    gt_table, cat_table, mech_table)
    return run_small(bgg_id.astype(jnp.int32), complexity.astype(jnp.int32),
                     bgg_table, cplx_norm, text)


# speedup vs baseline: 1.6677x; 1.6677x over previous
import functools

import jax
import jax.numpy as jnp
from jax import lax
from jax.experimental import pallas as pl
from jax.experimental.pallas import tpu as pltpu
from jax.experimental.pallas import tpu_sc as plsc

B = 16384
L = 20
D = 32
NC = 2
NS = 16
NW = NC * NS
BW = B // NW
CS = 32
NCHUNK = BW // CS
ROWS = CS * L
SUB = 128
NSUB = ROWS // SUB
ACC = 520


def _sc_text_body(gt_idx, cat_idx, mech_idx,
                  gt_table, cat_table, mech_table,
                  out,
                  tidx_v, rows0_v, rows1_v, pool_v,
                  gsrc0_v, gdst0_v, gsrc1_v, gdst1_v, recip_v,
                  acc_v, gsem0, gsem1, ssem0, ssem1):
    c = lax.axis_index("c")
    s = lax.axis_index("s")
    wid = c * NS + s
    base = wid * BW
    iota = lax.iota(jnp.int32, 16)

    def zero_pool(_unused):
        def zero_body(i, carry):
            zeros = jnp.zeros((16,), jnp.float32)
            pool_v[i, pl.ds(0, 16)] = zeros
            pool_v[i, pl.ds(16, 16)] = zeros
            return carry
        lax.fori_loop(0, BW, zero_body, 0)

    acc_base = s * ACC
    trash = acc_base + BW

    acc = acc_v
    for (idx_hbm, table, out_col) in (
            (gt_idx, gt_table, 0),
            (cat_idx, cat_table, D),
            (mech_idx, mech_table, 2 * D)):
        zero_pool(None)
        pltpu.sync_copy(pool_v, acc.at[pl.ds(s * ACC, BW)])
        pltpu.sync_copy(idx_hbm.at[pl.ds(base * L, BW * L)], tidx_v)

        def build(chunk, gsrc, gdst):
            for half in range(CS // 16):
                rows16 = chunk * CS + half * 16 + iota
                cnt = jnp.zeros((16,), jnp.int32)
                dst_ok = acc_base + rows16
                pos0 = rows16 * L
                for l in range(L):
                    v = plsc.load_gather(tidx_v, [pos0 + l])
                    m = v != 0
                    cnt = cnt + jnp.where(m, 1, 0).astype(jnp.int32)
                    dst = jnp.where(m, dst_ok, trash)
                    q = l * CS + half * 16
                    gsrc[q // SUB, pl.ds(q % SUB, 16)] = v
                    gdst[q // SUB, pl.ds(q % SUB, 16)] = dst
                r = 1.0 / jnp.maximum(cnt.astype(jnp.float32), 1.0)
                recip_v[pl.ds(chunk * CS + half * 16, 16)] = r

        def fire_gather(gsrc, rows, sem):
            for j in range(NSUB):
                pltpu.async_copy(table.at[gsrc.at[j]],
                                 rows.at[pl.ds(j * SUB, SUB)], sem)

        def drain(rows, sem):
            for j in range(NSUB):
                pltpu.make_async_copy(table.at[pl.ds(0, SUB)],
                                      rows.at[pl.ds(j * SUB, SUB)],
                                      sem).wait()

        def fire_scatter(rows, gdst, sem):
            for j in range(NSUB):
                pltpu.async_copy(rows.at[pl.ds(j * SUB, SUB)],
                                 acc.at[gdst.at[j]], sem, add=True)

        build(0, gsrc0_v, gdst0_v)
        fire_gather(gsrc0_v, rows0_v, gsem0)

        def chunk_body(k, carry):
            a = 2 * k

            @pl.when(k > 0)
            def _():
                drain(rows1_v, ssem1)
            build(a + 1, gsrc1_v, gdst1_v)
            fire_gather(gsrc1_v, rows1_v, gsem1)

            drain(rows0_v, gsem0)
            fire_scatter(rows0_v, gdst0_v, ssem0)
            drain(rows0_v, ssem0)

            @pl.when(k < NCHUNK // 2 - 1)
            def _():
                build(a + 2, gsrc0_v, gdst0_v)
                fire_gather(gsrc0_v, rows0_v, gsem0)

            drain(rows1_v, gsem1)
            fire_scatter(rows1_v, gdst1_v, ssem1)
            return carry
        lax.fori_loop(0, NCHUNK // 2, chunk_body, 0)
        drain(rows1_v, ssem1)

        pltpu.sync_copy(acc.at[pl.ds(acc_base, BW)], pool_v)

        def fix_body(g, carry):
            rv = recip_v[pl.ds(g * 16, 16)]
            for j in range(16):
                i = g * 16 + j
                r = rv[j]
                pool_v[i, pl.ds(0, 16)] = pool_v[i, pl.ds(0, 16)] * r
                pool_v[i, pl.ds(16, 16)] = pool_v[i, pl.ds(16, 16)] * r
            return carry
        lax.fori_loop(0, BW // 16, fix_body, 0)
        pltpu.sync_copy(pool_v, out.at[pl.ds(base, BW), pl.ds(out_col, D)])


def _sc_small_body(bgg_id, complexity, bgg_table, cplx_table, text, out,
                   pool_v, idxa_v, stage_v, gsem):
    c = lax.axis_index("c")
    s = lax.axis_index("s")
    wid = c * NS + s
    base = wid * BW

    pltpu.sync_copy(text.at[pl.ds(base, BW), pl.ds(0, 3 * D)], stage_v)
    pltpu.sync_copy(stage_v, out.at[pl.ds(base, BW), pl.ds(2 * D, 3 * D)])

    for (ids, table, col) in ((bgg_id, bgg_table, 0),
                              (complexity, cplx_table, D)):
        pltpu.sync_copy(ids.at[pl.ds(base, BW)], idxa_v)
        descs = []
        for j in range(BW // SUB):
            descs.append(pltpu.async_copy(
                table.at[idxa_v.at[pl.ds(j * SUB, SUB)]],
                pool_v.at[pl.ds(j * SUB, SUB)], gsem))
        for dsc in descs:
            dsc.wait()
        pltpu.sync_copy(pool_v, out.at[pl.ds(base, BW), pl.ds(col, D)])


@functools.partial(jax.jit, static_argnums=())
def kernel(bgg_id, complexity, game_type, category, mechanic,
           bgg_table, complexity_table, gt_table, cat_table, mech_table,
           norm_mean, norm_var):
    cplx_norm = ((complexity_table - norm_mean[None, :])
                 / jnp.sqrt(norm_var[None, :] + 1e-6))

    mesh = plsc.VectorSubcoreMesh(core_axis_name="c", subcore_axis_name="s")
    cp = pltpu.CompilerParams(
        use_tc_tiling_on_sc=False, needs_layout_passes=False)
    run_text = pl.kernel(
        _sc_text_body,
        out_type=jax.ShapeDtypeStruct((B, 3 * D), jnp.float32),
        mesh=mesh,
        compiler_params=cp,
        scratch_types=[
            pltpu.VMEM((BW * L,), jnp.int32),
            pltpu.VMEM((ROWS, D), jnp.float32),
            pltpu.VMEM((ROWS, D), jnp.float32),
            pltpu.VMEM((BW, D), jnp.float32),
            pltpu.VMEM((NSUB, SUB), jnp.int32),
            pltpu.VMEM((NSUB, SUB), jnp.int32),
            pltpu.VMEM((NSUB, SUB), jnp.int32),
            pltpu.VMEM((NSUB, SUB), jnp.int32),
            pltpu.VMEM((BW,), jnp.float32),
            pltpu.VMEM_SHARED((NS * ACC, D), jnp.float32),
            pltpu.SemaphoreType.DMA,
            pltpu.SemaphoreType.DMA,
            pltpu.SemaphoreType.DMA,
            pltpu.SemaphoreType.DMA,
        ],
    )
    run_small = pl.kernel(
        _sc_small_body,
        out_type=jax.ShapeDtypeStruct((B, 5 * D), jnp.float32),
        mesh=mesh,
        compiler_params=cp,
        scratch_types=[
            pltpu.VMEM((BW, D), jnp.float32),
            pltpu.VMEM((BW,), jnp.int32),
            pltpu.VMEM((BW, 3 * D), jnp.float32),
            pltpu.SemaphoreType.DMA,
        ],
    )
    text = run_text(game_type.astype(jnp.int32).reshape(-1),
                    category.astype(jnp.int32).reshape(-1),
                    mechanic.astype(jnp.int32).reshape(-1),
                    gt_table, cat_table, mech_table)
    return run_small(bgg_id.astype(jnp.int32), complexity.astype(jnp.int32),
                     bgg_table, cplx_norm, text)
